# baseline (speedup 1.0000x reference)
import jax
import jax.numpy as jnp
from jax.experimental import pallas as pl

_T, _D, _E, _K = 8192, 2048, 64, 8
_BT = 2048
_HB = _BT // 2


def _route_half(h_half, w):
    logits = jax.lax.dot_general(
        h_half, w,
        dimension_numbers=(((1,), (0,)), ((), ())),
        preferred_element_type=jnp.float32,
    )
    rowmax = jnp.max(logits, axis=-1, keepdims=True)
    e_full = jnp.exp(logits - rowmax)
    x = jnp.where(logits >= rowmax, -jnp.inf, logits)
    for _ in range(_K - 2):
        m = jnp.max(x, axis=-1, keepdims=True)
        x = jnp.where(x >= m, -jnp.inf, x)
    thr = jnp.max(x, axis=-1, keepdims=True)
    mask = logits >= thr
    e = jnp.where(mask, e_full, 0.0)
    probs = e / jnp.sum(e, axis=-1, keepdims=True)
    return mask.astype(jnp.int8), probs, logits


def _router_tile(h1_ref, h2_ref, w_ref, mask_ref, probs_ref, logits_ref):
    w = w_ref[...]
    m1, p1, l1 = _route_half(h1_ref[...], w)
    mask_ref[0:_HB, :] = m1
    probs_ref[0:_HB, :] = p1
    logits_ref[0:_HB, :] = l1
    m2, p2, l2 = _route_half(h2_ref[...], w)
    mask_ref[_HB:_BT, :] = m2
    probs_ref[_HB:_BT, :] = p2
    logits_ref[_HB:_BT, :] = l2


@jax.jit
def kernel(h, W):
    t, d = h.shape
    e = W.shape[1]
    grid = (t // _BT,)
    mask, probs, logits = pl.pallas_call(
        _router_tile,
        grid=grid,
        in_specs=[
            pl.BlockSpec((_HB, d), lambda i: (2 * i, 0)),
            pl.BlockSpec((_HB, d), lambda i: (2 * i + 1, 0)),
            pl.BlockSpec((d, e), lambda i: (0, 0)),
        ],
        out_specs=[
            pl.BlockSpec((_BT, e), lambda i: (i, 0)),
            pl.BlockSpec((_BT, e), lambda i: (i, 0)),
            pl.BlockSpec((_BT, e), lambda i: (i, 0)),
        ],
        out_shape=[
            jax.ShapeDtypeStruct((t, e), jnp.int8),
            jax.ShapeDtypeStruct((t, e), jnp.float32),
            jax.ShapeDtypeStruct((t, e), jnp.float32),
        ],
    )(h, h, W)
    return (mask.astype(bool), probs, logits, logits)


# speedup vs baseline: 1.0043x; 1.0043x over previous
import jax
import jax.numpy as jnp
from jax.experimental import pallas as pl

_T, _D, _E, _K = 8192, 2048, 64, 8
_BT = 2048


def _router_tile(h_ref, w_ref, mask_ref, probs_ref, logits_ref):
    logits = jax.lax.dot_general(
        h_ref[...], w_ref[...],
        dimension_numbers=(((1,), (0,)), ((), ())),
        preferred_element_type=jnp.float32,
    )
    rowmax = jnp.max(logits, axis=-1, keepdims=True)
    e_full = jnp.exp(logits - rowmax)
    x = jnp.where(logits >= rowmax, -jnp.inf, logits)
    for _ in range(_K - 2):
        m = jnp.max(x, axis=-1, keepdims=True)
        x = jnp.where(x >= m, -jnp.inf, x)
    thr = jnp.max(x, axis=-1, keepdims=True)
    mask = logits >= thr
    e = jnp.where(mask, e_full, 0.0)
    probs = e / jnp.sum(e, axis=-1, keepdims=True)
    mask_ref[...] = mask.astype(jnp.int8)
    probs_ref[...] = probs
    logits_ref[...] = logits


@jax.jit
def kernel(h, W):
    t, d = h.shape
    e = W.shape[1]
    grid = (t // _BT,)
    mask, probs, logits = pl.pallas_call(
        _router_tile,
        grid=grid,
        in_specs=[
            pl.BlockSpec((_BT, d), lambda i: (i, 0)),
            pl.BlockSpec((d, e), lambda i: (0, 0)),
        ],
        out_specs=[
            pl.BlockSpec((_BT, e), lambda i: (i, 0)),
            pl.BlockSpec((_BT, e), lambda i: (i, 0)),
            pl.BlockSpec((_BT, e), lambda i: (i, 0)),
        ],
        out_shape=[
            jax.ShapeDtypeStruct((t, e), jnp.int8),
            jax.ShapeDtypeStruct((t, e), jnp.float32),
            jax.ShapeDtypeStruct((t, e), jnp.float32),
        ],
    )(h, W)
    return (mask.astype(bool), probs, logits, logits)


# speedup vs baseline: 1.0057x; 1.0014x over previous
import jax
import jax.numpy as jnp
from jax.experimental import pallas as pl

_T, _D, _E, _K = 8192, 2048, 64, 8
_BT = 2048


def _router_tile(h_ref, w_ref, mask_ref, probs_ref, logits_ref):
    logits = jax.lax.dot_general(
        h_ref[...], w_ref[...],
        dimension_numbers=(((1,), (0,)), ((), ())),
        preferred_element_type=jnp.float32,
    )
    rowmax = jnp.max(logits, axis=-1, keepdims=True)
    e_full = jnp.exp(logits - rowmax)
    x = jnp.where(logits >= rowmax, -jnp.inf, logits)
    for _ in range(_K - 2):
        m = jnp.max(x, axis=-1, keepdims=True)
        x = jnp.where(x >= m, -jnp.inf, x)
    thr = jnp.max(x, axis=-1, keepdims=True)
    mask = logits >= thr
    e = jnp.where(mask, e_full, 0.0)
    probs = e / jnp.sum(e, axis=-1, keepdims=True)
    mask_ref[...] = mask.astype(jnp.int8)
    probs_ref[...] = probs
    logits_ref[...] = logits


@jax.jit
def kernel(h, W):
    t, d = h.shape
    e = W.shape[1]
    grid = (t // _BT,)
    mask, probs, logits = pl.pallas_call(
        _router_tile,
        grid=grid,
        in_specs=[
            pl.BlockSpec((_BT, d), lambda i: (i, 0)),
            pl.BlockSpec((d, e), lambda i: (0, 0)),
        ],
        out_specs=[
            pl.BlockSpec((_BT, e), lambda i: (i, 0)),
            pl.BlockSpec((_BT, e), lambda i: (i, 0)),
            pl.BlockSpec((_BT, e), lambda i: (i, 0)),
        ],
        out_shape=[
            jax.ShapeDtypeStruct((t, e), jnp.int8),
            jax.ShapeDtypeStruct((t, e), jnp.float32),
            jax.ShapeDtypeStruct((t, e), jnp.float32),
        ],
    )(h, W)
    return (mask.view(jnp.bool_), probs, logits, logits)
